# Initial kernel scaffold; baseline (speedup 1.0000x reference)
import jax
import jax.numpy as jnp
from jax.experimental import pallas as pl


def kernel(nodes, edges, edge_index, edge_pair):
    raise NotImplementedError("write your pallas kernel here")


# speedup vs baseline: 1.8682x; 1.8682x over previous
import functools

import jax
import jax.numpy as jnp
from jax import lax
from jax.experimental import pallas as pl
from jax.experimental.pallas import tpu as pltpu
from jax.experimental.pallas import tpu_sc as plsc

NC = 2
NS = 16
NW = NC * NS

_MESH = dict(core_axis_name="c", subcore_axis_name="s", num_cores=NC,
             num_subcores=NS)


def _phase1(edges, i0, n_nodes):
    E, D = edges.shape
    per_tile = E // NW
    C = 80
    n_chunks = per_tile // C
    EC = 200
    n_pool_chunks = n_nodes // EC
    nsub = D // 16

    @functools.partial(
        pl.kernel,
        out_type=jax.ShapeDtypeStruct((NC, n_nodes, D), jnp.float32),
        mesh=plsc.VectorSubcoreMesh(**_MESH),
        scratch_types=[
            pltpu.VMEM_SHARED((n_nodes, D), jnp.float32),
            pltpu.VMEM((C, D), jnp.float32),
            pltpu.VMEM((C,), jnp.int32),
            pltpu.VMEM((EC, D), jnp.float32),
        ],
    )
    def k1(edges_hbm, i0_hbm, out_hbm, pool_sh, rows_v, idx_v, exp_v):
        c = lax.axis_index("c")
        s = lax.axis_index("s")

        zero = jnp.zeros((16,), jnp.float32)

        def zbody(i, carry):
            r = i // nsub
            j = (i % nsub) * 16
            exp_v[r, pl.ds(j, 16)] = zero
            return carry

        lax.fori_loop(0, EC * nsub, zbody, 0)

        def zchunk(kk, carry):
            @pl.when(kk % NS == s)
            def _():
                pltpu.sync_copy(exp_v, pool_sh.at[pl.ds(kk * EC, EC)])
            return carry

        lax.fori_loop(0, n_pool_chunks, zchunk, 0)
        plsc.subcore_barrier()

        base = (c * NS + s) * per_tile

        def body(k, carry):
            off = base + k * C
            pltpu.sync_copy(i0_hbm.at[pl.ds(off, C)], idx_v)
            pltpu.sync_copy(edges_hbm.at[pl.ds(off, C)], rows_v)
            pltpu.sync_copy(rows_v, pool_sh.at[idx_v], add=True)
            return carry

        lax.fori_loop(0, n_chunks, body, 0)
        plsc.subcore_barrier()

        def echunk(kk, carry):
            @pl.when(kk % NS == s)
            def _():
                r0 = kk * EC
                pltpu.sync_copy(pool_sh.at[pl.ds(r0, EC)], exp_v)
                pltpu.sync_copy(exp_v, out_hbm.at[c, pl.ds(r0, EC)])
            return carry

        lax.fori_loop(0, n_pool_chunks, echunk, 0)

    return k1(edges, i0)


def _combine(partials):
    _, N, D = partials.shape
    BLK = 1000

    def body(p0_ref, p1_ref, o_ref):
        o_ref[...] = p0_ref[...] + p1_ref[...]

    return pl.pallas_call(
        body,
        grid=(N // BLK,),
        in_specs=[pl.BlockSpec((BLK, D), lambda i: (i, 0)),
                  pl.BlockSpec((BLK, D), lambda i: (i, 0))],
        out_specs=pl.BlockSpec((BLK, D), lambda i: (i, 0)),
        out_shape=jax.ShapeDtypeStruct((N, D), jnp.float32),
    )(partials[0], partials[1])


def _phase2(pool, edges, i1, ep):
    E, D = edges.shape
    per_tile = E // NW
    C = 80
    n_chunks = per_tile // C
    nsub = D // 16

    @functools.partial(
        pl.kernel,
        out_type=jax.ShapeDtypeStruct((E, D), jnp.float32),
        mesh=plsc.VectorSubcoreMesh(**_MESH),
        scratch_types=[
            pltpu.VMEM((C, D), jnp.float32),
            pltpu.VMEM((C, D), jnp.float32),
            pltpu.VMEM((C,), jnp.int32),
            pltpu.VMEM((C,), jnp.int32),
            pltpu.SemaphoreType.DMA,
            pltpu.SemaphoreType.DMA,
        ],
    )
    def k2(pool_hbm, edges_hbm, i1_hbm, ep_hbm, out_hbm,
           a_v, b_v, idx1_v, idxp_v, sem1, sem2):
        c = lax.axis_index("c")
        s = lax.axis_index("s")
        base = (c * NS + s) * per_tile

        def body(k, carry):
            off = base + k * C
            pltpu.sync_copy(i1_hbm.at[pl.ds(off, C)], idx1_v)
            pltpu.sync_copy(ep_hbm.at[pl.ds(off, C)], idxp_v)
            cp1 = pltpu.async_copy(pool_hbm.at[idx1_v], a_v, sem1)
            cp2 = pltpu.async_copy(edges_hbm.at[idxp_v], b_v, sem2)
            cp1.wait()
            cp2.wait()

            def sub_body(i, carry2):
                r = i // nsub
                j = (i % nsub) * 16
                a_v[r, pl.ds(j, 16)] = (a_v[r, pl.ds(j, 16)]
                                        - b_v[r, pl.ds(j, 16)])
                return carry2

            lax.fori_loop(0, C * nsub, sub_body, 0)
            pltpu.sync_copy(a_v, out_hbm.at[pl.ds(off, C)])
            return carry

        lax.fori_loop(0, n_chunks, body, 0)

    return k2(pool, edges, i1, ep)


def kernel(nodes, edges, edge_index, edge_pair):
    n_nodes = nodes.shape[0]
    i0 = edge_index[0]
    i1 = edge_index[1]
    ep = edge_pair[0]
    partials = _phase1(edges, i0, n_nodes)
    pool = _combine(partials)
    return _phase2(pool, edges, i1, ep)


# speedup vs baseline: 5.3718x; 2.8754x over previous
import functools

import jax
import jax.numpy as jnp
from jax import lax
from jax.experimental import pallas as pl
from jax.experimental.pallas import tpu as pltpu
from jax.experimental.pallas import tpu_sc as plsc

NC = 2
NS = 16
NW = NC * NS

C = 80
_MESH = dict(core_axis_name="c", subcore_axis_name="s", num_cores=NC,
             num_subcores=NS)


def _phase1(edges, i0r, n_nodes):
    E, D = edges.shape
    per_tile = E // NW
    n_chunks = per_tile // C
    EC = 80
    n_pool_chunks = n_nodes // EC
    nsub = D // 16

    @functools.partial(
        pl.kernel,
        out_type=jax.ShapeDtypeStruct((NC, n_nodes, D), jnp.float32),
        mesh=plsc.VectorSubcoreMesh(**_MESH),
        scratch_types=[
            pltpu.VMEM_SHARED((n_nodes, D), jnp.float32),
            pltpu.VMEM((C, D), jnp.float32),
            pltpu.VMEM((C, D), jnp.float32),
            pltpu.VMEM((n_chunks, C), jnp.int32),
            pltpu.VMEM((EC, D), jnp.float32),
            pltpu.SemaphoreType.DMA,
            pltpu.SemaphoreType.DMA,
            pltpu.SemaphoreType.DMA,
            pltpu.SemaphoreType.DMA,
        ],
    )
    def k1(edges_hbm, i0_hbm, out_hbm, pool_sh, rows0, rows1, idx_all, exp_v,
           ld0, ld1, sa0, sa1):
        c = lax.axis_index("c")
        s = lax.axis_index("s")
        wid = c * NS + s
        base = wid * per_tile

        pltpu.sync_copy(i0_hbm.at[wid], idx_all)

        zero = jnp.zeros((16,), jnp.float32)

        def zbody(i, carry):
            r = i // nsub
            j = (i % nsub) * 16
            exp_v[r, pl.ds(j, 16)] = zero
            return carry

        lax.fori_loop(0, EC * nsub, zbody, 0)

        def zchunk(kk, carry):
            @pl.when(kk % NS == s)
            def _():
                pltpu.sync_copy(exp_v, pool_sh.at[pl.ds(kk * EC, EC)])
            return carry

        lax.fori_loop(0, n_pool_chunks, zchunk, 0)
        plsc.subcore_barrier()

        rows = (rows0, rows1)
        ld = (ld0, ld1)
        sa = (sa0, sa1)

        def load(k, b):
            return pltpu.async_copy(
                edges_hbm.at[pl.ds(base + k * C, C)], rows[b], ld[b])

        load(0, 0)
        load(1, 1)

        def body2(g, carry):
            k0 = g * 2
            for b in (0, 1):
                k = k0 + b
                pltpu.make_async_copy(
                    edges_hbm.at[pl.ds(base + k * C, C)], rows[b],
                    ld[b]).wait()
                cp = pltpu.async_copy(rows[b], pool_sh.at[idx_all.at[k]],
                                      sa[b], add=True)
                cp.wait()

                @pl.when(k + 2 < n_chunks)
                def _():
                    load(k + 2, b)
            return carry

        lax.fori_loop(0, n_chunks // 2, body2, 0)
        if n_chunks % 2:
            k = n_chunks - 1
            b = k % 2
            pltpu.make_async_copy(
                edges_hbm.at[pl.ds(base + k * C, C)], rows[b], ld[b]).wait()
            pltpu.async_copy(rows[b], pool_sh.at[idx_all.at[k]], sa[b],
                             add=True).wait()
        plsc.subcore_barrier()

        def echunk(kk, carry):
            @pl.when(kk % NS == s)
            def _():
                r0 = kk * EC
                pltpu.sync_copy(pool_sh.at[pl.ds(r0, EC)], exp_v)
                pltpu.sync_copy(exp_v, out_hbm.at[c, pl.ds(r0, EC)])
            return carry

        lax.fori_loop(0, n_pool_chunks, echunk, 0)

    return k1(edges, i0r)


def _combine(partials):
    _, N, D = partials.shape
    BLK = 1000

    def body(p0_ref, p1_ref, o_ref):
        o_ref[...] = p0_ref[...] + p1_ref[...]

    return pl.pallas_call(
        body,
        grid=(N // BLK,),
        in_specs=[pl.BlockSpec((BLK, D), lambda i: (i, 0)),
                  pl.BlockSpec((BLK, D), lambda i: (i, 0))],
        out_specs=pl.BlockSpec((BLK, D), lambda i: (i, 0)),
        out_shape=jax.ShapeDtypeStruct((N, D), jnp.float32),
    )(partials[0], partials[1])


def _phase2(pool, edges, i1r, epr):
    E, D = edges.shape
    per_tile = E // NW
    n_chunks = per_tile // C
    nsub = D // 16

    @functools.partial(
        pl.kernel,
        out_type=jax.ShapeDtypeStruct((E, D), jnp.float32),
        mesh=plsc.VectorSubcoreMesh(**_MESH),
        scratch_types=[
            pltpu.VMEM((C, D), jnp.float32),
            pltpu.VMEM((C, D), jnp.float32),
            pltpu.VMEM((C, D), jnp.float32),
            pltpu.VMEM((C, D), jnp.float32),
            pltpu.VMEM((C, D), jnp.float32),
            pltpu.VMEM((C, D), jnp.float32),
            pltpu.VMEM((n_chunks, C), jnp.int32),
            pltpu.VMEM((n_chunks, C), jnp.int32),
            pltpu.SemaphoreType.DMA,
            pltpu.SemaphoreType.DMA,
            pltpu.SemaphoreType.DMA,
            pltpu.SemaphoreType.DMA,
        ],
    )
    def k2(pool_hbm, edges_hbm, i1_hbm, ep_hbm, out_hbm,
           a0, a1, b0, b1, o0, o1, idx1_all, idxp_all, g0, g1, st0, st1):
        c = lax.axis_index("c")
        s = lax.axis_index("s")
        wid = c * NS + s
        base = wid * per_tile

        pltpu.sync_copy(i1_hbm.at[wid], idx1_all)
        pltpu.sync_copy(ep_hbm.at[wid], idxp_all)

        A = (a0, a1)
        B = (b0, b1)
        O = (o0, o1)
        g = (g0, g1)
        st = (st0, st1)

        def gathers(k, b):
            pltpu.async_copy(pool_hbm.at[idx1_all.at[k]], A[b], g[b])
            pltpu.async_copy(edges_hbm.at[idxp_all.at[k]], B[b], g[b])

        def wait_gathers(k, b):
            pltpu.make_async_copy(pool_hbm.at[idx1_all.at[k]], A[b],
                                  g[b]).wait()
            pltpu.make_async_copy(edges_hbm.at[idxp_all.at[k]], B[b],
                                  g[b]).wait()

        def store(k, b):
            return pltpu.async_copy(O[b], out_hbm.at[pl.ds(base + k * C, C)],
                                    st[b])

        def wait_store(k, b):
            pltpu.make_async_copy(O[b], out_hbm.at[pl.ds(base + k * C, C)],
                                  st[b]).wait()

        def subtract(b):
            ab, bb, ob = A[b], B[b], O[b]

            def sbody(r, carry):
                for jj in range(nsub):
                    j = jj * 16
                    ob[r, pl.ds(j, 16)] = (ab[r, pl.ds(j, 16)]
                                           - bb[r, pl.ds(j, 16)])
                return carry

            lax.fori_loop(0, C, sbody, 0)

        gathers(0, 0)
        gathers(1, 1)

        def body2(gidx, carry):
            k0 = gidx * 2
            for b in (0, 1):
                k = k0 + b
                wait_gathers(k, b)

                @pl.when(k > 1)
                def _():
                    wait_store(k - 2, b)

                subtract(b)
                store(k, b)

                @pl.when(k + 2 < n_chunks)
                def _():
                    gathers(k + 2, b)
            return carry

        lax.fori_loop(0, n_chunks // 2, body2, 0)
        if n_chunks % 2:
            k = n_chunks - 1
            b = k % 2
            wait_gathers(k, b)
            wait_store(k - 2, b)
            subtract(b)
            store(k, b)
            wait_store(k - 1, 1 - b)
            wait_store(k, b)
        else:
            wait_store(n_chunks - 2, 0)
            wait_store(n_chunks - 1, 1)

    return k2(pool, edges, i1r, epr)


def kernel(nodes, edges, edge_index, edge_pair):
    n_nodes = nodes.shape[0]
    E = edges.shape[0]
    per_tile = E // NW
    n_chunks = per_tile // C
    i0r = edge_index[0].reshape(NW, n_chunks, C)
    i1r = edge_index[1].reshape(NW, n_chunks, C)
    epr = edge_pair[0].reshape(NW, n_chunks, C)
    partials = _phase1(edges, i0r, n_nodes)
    pool = _combine(partials)
    return _phase2(pool, edges, i1r, epr)
